# baseline (speedup 1.0000x reference)
import jax
import jax.numpy as jnp
from jax import lax
from jax.experimental import pallas as pl
from jax.experimental.pallas import tpu as pltpu
from jax.experimental.pallas import tpu_sc as plsc

N_NODES = 10000
N_PAD = 10240
D = 128
N_EDGES = 320000
CHUNK = 128
NS = 16
T = 158
N_CHUNKS = NS * T
E_PAD = N_CHUNKS * CHUNK
ROWS_PER_TILE = N_PAD // NS


def _sc_body(xa_hbm, idx_hbm, zeros2_hbm, zeros1_hbm, feat_out, deg_out,
             idx_v, rows_v, hist_v, acc_sh, gsem0):
    c = lax.axis_index("c")
    s = lax.axis_index("s")
    pltpu.sync_copy(zeros2_hbm, acc_sh.at[pl.ds(s * ROWS_PER_TILE, ROWS_PER_TILE)])
    pltpu.sync_copy(zeros1_hbm, hist_v)
    base = s * T

    ones16 = jnp.ones((16,), jnp.float32)

    def hist_chunk():
        for k in range(CHUNK // 16):
            idx16 = idx_v[1, pl.ds(k * 16, 16)]
            plsc.addupdate_scatter(hist_v, [idx16], ones16)

    plsc.subcore_barrier()

    def body(t, carry):
        pltpu.sync_copy(idx_hbm.at[c, base + t], idx_v)
        gather = pltpu.async_copy(xa_hbm.at[idx_v.at[0]], rows_v, gsem0)
        hist_chunk()
        gather.wait()
        pltpu.sync_copy(rows_v, acc_sh.at[idx_v.at[1]], add=True)
        return carry

    lax.fori_loop(0, jnp.maximum(T, s), body, 0)

    pltpu.sync_copy(hist_v, deg_out.at[c, s])
    plsc.subcore_barrier()
    pltpu.sync_copy(acc_sh.at[pl.ds(s * ROWS_PER_TILE, ROWS_PER_TILE)],
                    feat_out.at[c, pl.ds(s * ROWS_PER_TILE, ROWS_PER_TILE)])


_sc_call = pl.kernel(
    _sc_body,
    out_type=(
        jax.ShapeDtypeStruct((2, N_PAD, D), jnp.float32),
        jax.ShapeDtypeStruct((2, NS, N_PAD), jnp.float32),
    ),
    mesh=plsc.VectorSubcoreMesh(core_axis_name="c", subcore_axis_name="s"),
    compiler_params=pltpu.CompilerParams(needs_layout_passes=False),
    scratch_types=[
        pltpu.VMEM((2, CHUNK), jnp.int32),
        pltpu.VMEM((CHUNK, D), jnp.float32),
        pltpu.VMEM((N_PAD,), jnp.float32),
        pltpu.VMEM_SHARED((N_PAD, D), jnp.float32),
        pltpu.SemaphoreType.DMA,
    ],
)


BLK = 1024


def _tc_body(x_ref, aF_ref, dF_ref, aR_ref, dR_ref,
             wl1_ref, wr1_ref, wl2_ref, wr2_ref, b1_ref, b2_ref, out_ref):
    x = x_ref[...]
    ones_col = jnp.ones((NS, 1), jnp.float32)
    dn = (((0,), (0,)), ((), ()))
    degF = lax.dot_general(dF_ref[...], ones_col, dn,
                           preferred_element_type=jnp.float32)
    degR = lax.dot_general(dR_ref[...], ones_col, dn,
                           preferred_element_type=jnp.float32)
    meanF = aF_ref[...] * (1.0 / jnp.maximum(degF, 1.0))
    meanR = aR_ref[...] * (1.0 / jnp.maximum(degR, 1.0))
    outF = (jnp.dot(meanF, wl1_ref[...], preferred_element_type=jnp.float32)
            + b1_ref[...]
            + jnp.dot(x, wr1_ref[...], preferred_element_type=jnp.float32))
    outR = (jnp.dot(meanR, wl2_ref[...], preferred_element_type=jnp.float32)
            + b2_ref[...]
            + jnp.dot(x, wr2_ref[...], preferred_element_type=jnp.float32))
    out_ref[:, :D] = outF
    out_ref[:, D:] = outR


def _tc_call(x, aF, dFt, aR, dRt, wl1t, wr1t, wl2t, wr2t, b1, b2):
    grid = pl.cdiv(N_NODES, BLK)
    row_spec = pl.BlockSpec((BLK, D), lambda i: (i, 0))
    deg_spec = pl.BlockSpec((NS, BLK), lambda i: (0, i))
    full_spec = lambda a, b: pl.BlockSpec((a, b), lambda i: (0, 0))
    return pl.pallas_call(
        _tc_body,
        grid=(grid,),
        in_specs=[
            row_spec, row_spec, deg_spec, row_spec, deg_spec,
            full_spec(D, D), full_spec(D, D), full_spec(D, D), full_spec(D, D),
            full_spec(1, D), full_spec(1, D),
        ],
        out_specs=pl.BlockSpec((BLK, 2 * D), lambda i: (i, 0)),
        out_shape=jax.ShapeDtypeStruct((N_NODES, 2 * D), jnp.float32),
    )(x, aF, dFt, aR, dRt, wl1t, wr1t, wl2t, wr2t, b1, b2)


@jax.jit
def kernel(x, edge_index, W_l1, b_l1, W_r1, W_l2, b_l2, W_r2):
    ei = edge_index.astype(jnp.int32)
    src, dst = ei[0], ei[1]
    npad = E_PAD - N_EDGES
    gpad = jnp.zeros((npad,), jnp.int32)
    spad = N_NODES + jnp.arange(npad, dtype=jnp.int32) % (N_PAD - N_NODES)
    def build(g, sc):
        g = jnp.concatenate([g, gpad]).reshape(N_CHUNKS, CHUNK)
        sc = jnp.concatenate([sc, spad]).reshape(N_CHUNKS, CHUNK)
        return jnp.stack([g, sc], 1)
    idx = jnp.stack([build(src, dst), build(dst, src)])
    zeros2 = jnp.zeros((ROWS_PER_TILE, D), jnp.float32)
    zeros1 = jnp.zeros((N_PAD,), jnp.float32)
    feat, deg = _sc_call(x, idx, zeros2, zeros1)
    aF = feat[0, :N_NODES]
    aR = feat[1, :N_NODES]
    return _tc_call(x, aF, deg[0], aR, deg[1],
                    W_l1.T, W_r1.T, W_l2.T, W_r2.T,
                    b_l1.reshape(1, D), b_l2.reshape(1, D))


# speedup vs baseline: 1.4470x; 1.4470x over previous
import jax
import jax.numpy as jnp
from jax import lax
from jax.experimental import pallas as pl
from jax.experimental.pallas import tpu as pltpu
from jax.experimental.pallas import tpu_sc as plsc

N_NODES = 10000
N_PAD = 10240
D = 128
N_EDGES = 320000
CHUNK = 128
N_CHUNKS = N_EDGES // CHUNK
NS = 16
ROWS_PER_TILE = N_PAD // NS
CH_BASE = N_CHUNKS // NS
CH_REM = N_CHUNKS % NS


def _sc_body(xa_hbm, idx_hbm, zeros2_hbm, zeros1_hbm, feat_out, deg_out,
             idx_v, rows_v, hist_v, acc_sh, sem):
    c = lax.axis_index("c")
    s = lax.axis_index("s")
    pltpu.sync_copy(zeros2_hbm, acc_sh.at[pl.ds(s * ROWS_PER_TILE, ROWS_PER_TILE)])
    pltpu.sync_copy(zeros1_hbm, hist_v)
    plsc.subcore_barrier()

    lo = s * CH_BASE + jnp.minimum(s, CH_REM)
    n = CH_BASE + jnp.where(s < CH_REM, 1, 0)
    ones16 = jnp.ones((16,), jnp.float32)

    def body(j, carry):
        pltpu.sync_copy(idx_hbm.at[c, j], idx_v)
        gather = pltpu.async_copy(xa_hbm.at[idx_v.at[0]], rows_v, sem)
        for k in range(CHUNK // 16):
            idx16 = idx_v[1, pl.ds(k * 16, 16)]
            plsc.addupdate_scatter(hist_v, [idx16], ones16)
        gather.wait()
        pltpu.sync_copy(rows_v, acc_sh.at[idx_v.at[1]], add=True)
        return carry

    lax.fori_loop(lo, lo + n, body, 0)
    pltpu.sync_copy(hist_v, deg_out.at[c, s])
    plsc.subcore_barrier()
    pltpu.sync_copy(acc_sh.at[pl.ds(s * ROWS_PER_TILE, ROWS_PER_TILE)],
                    feat_out.at[c, pl.ds(s * ROWS_PER_TILE, ROWS_PER_TILE)])


_sc_call = pl.kernel(
    _sc_body,
    out_type=(
        jax.ShapeDtypeStruct((2, N_PAD, D), jnp.float32),
        jax.ShapeDtypeStruct((2, NS, N_PAD), jnp.float32),
    ),
    mesh=plsc.VectorSubcoreMesh(core_axis_name="c", subcore_axis_name="s"),
    compiler_params=pltpu.CompilerParams(needs_layout_passes=False),
    scratch_types=[
        pltpu.VMEM((2, CHUNK), jnp.int32),
        pltpu.VMEM((CHUNK, D), jnp.float32),
        pltpu.VMEM((N_PAD,), jnp.float32),
        pltpu.VMEM_SHARED((N_PAD, D), jnp.float32),
        pltpu.SemaphoreType.DMA,
    ],
)


BLK = 1024


def _tc_body(x_ref, aF_ref, dF_ref, aR_ref, dR_ref,
             wl1_ref, wr1_ref, wl2_ref, wr2_ref, b1_ref, b2_ref, out_ref):
    x = x_ref[...]
    ones_col = jnp.ones((NS, 1), jnp.float32)
    dn = (((0,), (0,)), ((), ()))
    degF = lax.dot_general(dF_ref[...], ones_col, dn,
                           preferred_element_type=jnp.float32)
    degR = lax.dot_general(dR_ref[...], ones_col, dn,
                           preferred_element_type=jnp.float32)
    meanF = aF_ref[...] * (1.0 / jnp.maximum(degF, 1.0))
    meanR = aR_ref[...] * (1.0 / jnp.maximum(degR, 1.0))
    outF = (jnp.dot(meanF, wl1_ref[...], preferred_element_type=jnp.float32)
            + b1_ref[...]
            + jnp.dot(x, wr1_ref[...], preferred_element_type=jnp.float32))
    outR = (jnp.dot(meanR, wl2_ref[...], preferred_element_type=jnp.float32)
            + b2_ref[...]
            + jnp.dot(x, wr2_ref[...], preferred_element_type=jnp.float32))
    out_ref[:, :D] = outF
    out_ref[:, D:] = outR


def _tc_call(x, aF, dFt, aR, dRt, wl1t, wr1t, wl2t, wr2t, b1, b2):
    grid = pl.cdiv(N_NODES, BLK)
    row_spec = pl.BlockSpec((BLK, D), lambda i: (i, 0))
    deg_spec = pl.BlockSpec((NS, BLK), lambda i: (0, i))
    full_spec = lambda a, b: pl.BlockSpec((a, b), lambda i: (0, 0))
    return pl.pallas_call(
        _tc_body,
        grid=(grid,),
        in_specs=[
            row_spec, row_spec, deg_spec, row_spec, deg_spec,
            full_spec(D, D), full_spec(D, D), full_spec(D, D), full_spec(D, D),
            full_spec(1, D), full_spec(1, D),
        ],
        out_specs=pl.BlockSpec((BLK, 2 * D), lambda i: (i, 0)),
        out_shape=jax.ShapeDtypeStruct((N_NODES, 2 * D), jnp.float32),
    )(x, aF, dFt, aR, dRt, wl1t, wr1t, wl2t, wr2t, b1, b2)


@jax.jit
def kernel(x, edge_index, W_l1, b_l1, W_r1, W_l2, b_l2, W_r2):
    ei = edge_index.astype(jnp.int32)
    src, dst = ei[0], ei[1]
    fwd = jnp.stack([src.reshape(N_CHUNKS, CHUNK), dst.reshape(N_CHUNKS, CHUNK)], 1)
    rev = fwd[:, ::-1]
    idx = jnp.stack([fwd, rev])
    zeros2 = jnp.zeros((ROWS_PER_TILE, D), jnp.float32)
    zeros1 = jnp.zeros((N_PAD,), jnp.float32)
    feat, deg = _sc_call(x, idx, zeros2, zeros1)
    aF = feat[0, :N_NODES]
    aR = feat[1, :N_NODES]
    return _tc_call(x, aF, deg[0], aR, deg[1],
                    W_l1.T, W_r1.T, W_l2.T, W_r2.T,
                    b_l1.reshape(1, D), b_l2.reshape(1, D))


# speedup vs baseline: 2.2621x; 1.5634x over previous
import jax
import jax.numpy as jnp
from jax import lax
from jax.experimental import pallas as pl
from jax.experimental.pallas import tpu as pltpu
from jax.experimental.pallas import tpu_sc as plsc

N_NODES = 10000
N_PAD = 10240
D = 128
N_EDGES = 320000
CHUNK = 128
N_CHUNKS = N_EDGES // CHUNK
NS = 16
ROWS_PER_TILE = N_PAD // NS
CH_BASE = N_CHUNKS // NS
CH_REM = N_CHUNKS % NS


def _sc_body(xa_hbm, idx_hbm, zeros2_hbm, zeros1_hbm, feat_out, deg_out,
             idx_v, rows_v, hist_v, acc_sh, gsem0, gsem1):
    c = lax.axis_index("c")
    s = lax.axis_index("s")
    pltpu.sync_copy(zeros2_hbm, acc_sh.at[pl.ds(s * ROWS_PER_TILE, ROWS_PER_TILE)])
    pltpu.sync_copy(zeros1_hbm, hist_v)

    lo = s * CH_BASE + jnp.minimum(s, CH_REM)
    n = CH_BASE + jnp.where(s < CH_REM, 1, 0)
    ones16 = jnp.ones((16,), jnp.float32)

    def hist_chunk(b):
        for k in range(CHUNK // 16):
            idx16 = idx_v[b, 1, pl.ds(k * 16, 16)]
            plsc.addupdate_scatter(hist_v, [idx16], ones16)

    def start_gather(b):
        sem = gsem0 if b == 0 else gsem1
        pltpu.async_copy(xa_hbm.at[idx_v.at[b, 0]], rows_v.at[b], sem)

    def drain(b):
        sem = gsem0 if b == 0 else gsem1
        pltpu.make_async_copy(
            xa_hbm.at[idx_v.at[b, 0]], rows_v.at[b], sem).wait()
        pltpu.sync_copy(rows_v.at[b], acc_sh.at[idx_v.at[b, 1]], add=True)
        hist_chunk(b)

    pltpu.sync_copy(idx_hbm.at[c, lo], idx_v.at[lo & 1])

    @pl.when((lo & 1) == 0)
    def _():
        start_gather(0)

    @pl.when((lo & 1) == 1)
    def _():
        start_gather(1)

    plsc.subcore_barrier()

    def body(j, carry):
        par = j & 1
        pltpu.sync_copy(idx_hbm.at[c, j], idx_v.at[par])

        @pl.when(par == 0)
        def _():
            start_gather(0)
            drain(1)

        @pl.when(par == 1)
        def _():
            start_gather(1)
            drain(0)

        return carry

    lax.fori_loop(lo + 1, lo + n, body, 0)

    lpar = (lo + n - 1) & 1

    @pl.when(lpar == 0)
    def _():
        drain(0)

    @pl.when(lpar == 1)
    def _():
        drain(1)

    pltpu.sync_copy(hist_v, deg_out.at[c, s])
    plsc.subcore_barrier()
    pltpu.sync_copy(acc_sh.at[pl.ds(s * ROWS_PER_TILE, ROWS_PER_TILE)],
                    feat_out.at[c, pl.ds(s * ROWS_PER_TILE, ROWS_PER_TILE)])


_sc_call = pl.kernel(
    _sc_body,
    out_type=(
        jax.ShapeDtypeStruct((2, N_PAD, D), jnp.float32),
        jax.ShapeDtypeStruct((2, NS, N_PAD), jnp.float32),
    ),
    mesh=plsc.VectorSubcoreMesh(core_axis_name="c", subcore_axis_name="s"),
    compiler_params=pltpu.CompilerParams(needs_layout_passes=False),
    scratch_types=[
        pltpu.VMEM((2, 2, CHUNK), jnp.int32),
        pltpu.VMEM((2, CHUNK, D), jnp.float32),
        pltpu.VMEM((N_PAD,), jnp.float32),
        pltpu.VMEM_SHARED((N_PAD, D), jnp.float32),
        pltpu.SemaphoreType.DMA,
        pltpu.SemaphoreType.DMA,
    ],
)


BLK = 1024


def _tc_body(x_ref, aF_ref, dF_ref, aR_ref, dR_ref,
             wl1_ref, wr1_ref, wl2_ref, wr2_ref, b1_ref, b2_ref, out_ref):
    x = x_ref[...]
    ones_col = jnp.ones((NS, 1), jnp.float32)
    dn = (((0,), (0,)), ((), ()))
    degF = lax.dot_general(dF_ref[...], ones_col, dn,
                           preferred_element_type=jnp.float32)
    degR = lax.dot_general(dR_ref[...], ones_col, dn,
                           preferred_element_type=jnp.float32)
    meanF = aF_ref[...] * (1.0 / jnp.maximum(degF, 1.0))
    meanR = aR_ref[...] * (1.0 / jnp.maximum(degR, 1.0))
    outF = (jnp.dot(meanF, wl1_ref[...], preferred_element_type=jnp.float32)
            + b1_ref[...]
            + jnp.dot(x, wr1_ref[...], preferred_element_type=jnp.float32))
    outR = (jnp.dot(meanR, wl2_ref[...], preferred_element_type=jnp.float32)
            + b2_ref[...]
            + jnp.dot(x, wr2_ref[...], preferred_element_type=jnp.float32))
    out_ref[:, :D] = outF
    out_ref[:, D:] = outR


def _tc_call(x, aF, dFt, aR, dRt, wl1t, wr1t, wl2t, wr2t, b1, b2):
    grid = pl.cdiv(N_NODES, BLK)
    row_spec = pl.BlockSpec((BLK, D), lambda i: (i, 0))
    deg_spec = pl.BlockSpec((NS, BLK), lambda i: (0, i))
    full_spec = lambda a, b: pl.BlockSpec((a, b), lambda i: (0, 0))
    return pl.pallas_call(
        _tc_body,
        grid=(grid,),
        in_specs=[
            row_spec, row_spec, deg_spec, row_spec, deg_spec,
            full_spec(D, D), full_spec(D, D), full_spec(D, D), full_spec(D, D),
            full_spec(1, D), full_spec(1, D),
        ],
        out_specs=pl.BlockSpec((BLK, 2 * D), lambda i: (i, 0)),
        out_shape=jax.ShapeDtypeStruct((N_NODES, 2 * D), jnp.float32),
    )(x, aF, dFt, aR, dRt, wl1t, wr1t, wl2t, wr2t, b1, b2)


@jax.jit
def kernel(x, edge_index, W_l1, b_l1, W_r1, W_l2, b_l2, W_r2):
    ei = edge_index.astype(jnp.int32)
    src, dst = ei[0], ei[1]
    fwd = jnp.stack([src.reshape(N_CHUNKS, CHUNK), dst.reshape(N_CHUNKS, CHUNK)], 1)
    rev = fwd[:, ::-1]
    idx = jnp.stack([fwd, rev])
    zeros2 = jnp.zeros((ROWS_PER_TILE, D), jnp.float32)
    zeros1 = jnp.zeros((N_PAD,), jnp.float32)
    feat, deg = _sc_call(x, idx, zeros2, zeros1)
    aF = feat[0, :N_NODES]
    aR = feat[1, :N_NODES]
    return _tc_call(x, aF, deg[0], aR, deg[1],
                    W_l1.T, W_r1.T, W_l2.T, W_r2.T,
                    b_l1.reshape(1, D), b_l2.reshape(1, D))


# speedup vs baseline: 2.5551x; 1.1295x over previous
import jax
import jax.numpy as jnp
from jax import lax
from jax.experimental import pallas as pl
from jax.experimental.pallas import tpu as pltpu
from jax.experimental.pallas import tpu_sc as plsc

N_NODES = 10000
N_PAD = 10240
D = 128
N_EDGES = 320000
CHUNK = 128
N_CHUNKS = N_EDGES // CHUNK
NS = 16
ROWS_PER_TILE = N_PAD // NS
CH_BASE = N_CHUNKS // NS
CH_REM = N_CHUNKS % NS


def _sc_body(xa_hbm, idx_hbm, zeros2_hbm, zeros1_hbm, feat_out, deg_out,
             idx_v, rows_v, hist_v, acc_sh, gsem0, gsem1, ssem0, ssem1):
    c = lax.axis_index("c")
    s = lax.axis_index("s")
    pltpu.sync_copy(zeros2_hbm, acc_sh.at[pl.ds(s * ROWS_PER_TILE, ROWS_PER_TILE)])
    pltpu.sync_copy(zeros1_hbm, hist_v)

    lo = s * CH_BASE + jnp.minimum(s, CH_REM)
    n = CH_BASE + jnp.where(s < CH_REM, 1, 0)
    ones16 = jnp.ones((16,), jnp.float32)

    def hist_chunk(b):
        for k in range(CHUNK // 16):
            idx16 = idx_v[b, 1, pl.ds(k * 16, 16)]
            plsc.addupdate_scatter(hist_v, [idx16], ones16)

    def start_gather(b):
        sem = gsem0 if b == 0 else gsem1
        pltpu.async_copy(xa_hbm.at[idx_v.at[b, 0]], rows_v.at[b], sem)

    def wait_scatter(b):
        sem = ssem0 if b == 0 else ssem1
        pltpu.make_async_copy(
            rows_v.at[b], acc_sh.at[idx_v.at[b, 1]], sem).wait()

    def drain(b):
        gsem = gsem0 if b == 0 else gsem1
        ssem = ssem0 if b == 0 else ssem1
        pltpu.make_async_copy(
            xa_hbm.at[idx_v.at[b, 0]], rows_v.at[b], gsem).wait()
        pltpu.async_copy(rows_v.at[b], acc_sh.at[idx_v.at[b, 1]], ssem,
                         add=True)
        hist_chunk(b)

    pltpu.sync_copy(idx_hbm.at[c, lo], idx_v.at[lo & 1])

    @pl.when((lo & 1) == 0)
    def _():
        start_gather(0)

    @pl.when((lo & 1) == 1)
    def _():
        start_gather(1)

    plsc.subcore_barrier()

    pltpu.sync_copy(idx_hbm.at[c, lo + 1], idx_v.at[(lo + 1) & 1])

    @pl.when(((lo + 1) & 1) == 0)
    def _():
        start_gather(0)
        drain(1)

    @pl.when(((lo + 1) & 1) == 1)
    def _():
        start_gather(1)
        drain(0)

    def body(j, carry):
        par = j & 1
        pltpu.sync_copy(idx_hbm.at[c, j], idx_v.at[par])

        @pl.when(par == 0)
        def _():
            wait_scatter(0)
            start_gather(0)
            drain(1)

        @pl.when(par == 1)
        def _():
            wait_scatter(1)
            start_gather(1)
            drain(0)

        return carry

    lax.fori_loop(lo + 2, lo + n, body, 0)

    lpar = (lo + n - 1) & 1

    @pl.when(lpar == 0)
    def _():
        drain(0)
        wait_scatter(1)
        wait_scatter(0)

    @pl.when(lpar == 1)
    def _():
        drain(1)
        wait_scatter(0)
        wait_scatter(1)

    pltpu.sync_copy(hist_v, deg_out.at[c, s])
    plsc.subcore_barrier()
    pltpu.sync_copy(acc_sh.at[pl.ds(s * ROWS_PER_TILE, ROWS_PER_TILE)],
                    feat_out.at[c, pl.ds(s * ROWS_PER_TILE, ROWS_PER_TILE)])


_sc_call = pl.kernel(
    _sc_body,
    out_type=(
        jax.ShapeDtypeStruct((2, N_PAD, D), jnp.float32),
        jax.ShapeDtypeStruct((2, NS, N_PAD), jnp.float32),
    ),
    mesh=plsc.VectorSubcoreMesh(core_axis_name="c", subcore_axis_name="s"),
    compiler_params=pltpu.CompilerParams(needs_layout_passes=False),
    scratch_types=[
        pltpu.VMEM((2, 2, CHUNK), jnp.int32),
        pltpu.VMEM((2, CHUNK, D), jnp.float32),
        pltpu.VMEM((N_PAD,), jnp.float32),
        pltpu.VMEM_SHARED((N_PAD, D), jnp.float32),
        pltpu.SemaphoreType.DMA,
        pltpu.SemaphoreType.DMA,
        pltpu.SemaphoreType.DMA,
        pltpu.SemaphoreType.DMA,
    ],
)


BLK = 1024


def _tc_body(x_ref, aF_ref, dF_ref, aR_ref, dR_ref,
             wl1_ref, wr1_ref, wl2_ref, wr2_ref, b1_ref, b2_ref, out_ref):
    x = x_ref[...]
    ones_col = jnp.ones((NS, 1), jnp.float32)
    dn = (((0,), (0,)), ((), ()))
    degF = lax.dot_general(dF_ref[...], ones_col, dn,
                           preferred_element_type=jnp.float32)
    degR = lax.dot_general(dR_ref[...], ones_col, dn,
                           preferred_element_type=jnp.float32)
    meanF = aF_ref[...] * (1.0 / jnp.maximum(degF, 1.0))
    meanR = aR_ref[...] * (1.0 / jnp.maximum(degR, 1.0))
    outF = (jnp.dot(meanF, wl1_ref[...], preferred_element_type=jnp.float32)
            + b1_ref[...]
            + jnp.dot(x, wr1_ref[...], preferred_element_type=jnp.float32))
    outR = (jnp.dot(meanR, wl2_ref[...], preferred_element_type=jnp.float32)
            + b2_ref[...]
            + jnp.dot(x, wr2_ref[...], preferred_element_type=jnp.float32))
    out_ref[:, :D] = outF
    out_ref[:, D:] = outR


def _tc_call(x, aF, dFt, aR, dRt, wl1t, wr1t, wl2t, wr2t, b1, b2):
    grid = pl.cdiv(N_NODES, BLK)
    row_spec = pl.BlockSpec((BLK, D), lambda i: (i, 0))
    deg_spec = pl.BlockSpec((NS, BLK), lambda i: (0, i))
    full_spec = lambda a, b: pl.BlockSpec((a, b), lambda i: (0, 0))
    return pl.pallas_call(
        _tc_body,
        grid=(grid,),
        in_specs=[
            row_spec, row_spec, deg_spec, row_spec, deg_spec,
            full_spec(D, D), full_spec(D, D), full_spec(D, D), full_spec(D, D),
            full_spec(1, D), full_spec(1, D),
        ],
        out_specs=pl.BlockSpec((BLK, 2 * D), lambda i: (i, 0)),
        out_shape=jax.ShapeDtypeStruct((N_NODES, 2 * D), jnp.float32),
    )(x, aF, dFt, aR, dRt, wl1t, wr1t, wl2t, wr2t, b1, b2)


@jax.jit
def kernel(x, edge_index, W_l1, b_l1, W_r1, W_l2, b_l2, W_r2):
    ei = edge_index.astype(jnp.int32)
    src, dst = ei[0], ei[1]
    fwd = jnp.stack([src.reshape(N_CHUNKS, CHUNK), dst.reshape(N_CHUNKS, CHUNK)], 1)
    rev = fwd[:, ::-1]
    idx = jnp.stack([fwd, rev])
    zeros2 = jnp.zeros((ROWS_PER_TILE, D), jnp.float32)
    zeros1 = jnp.zeros((N_PAD,), jnp.float32)
    feat, deg = _sc_call(x, idx, zeros2, zeros1)
    aF = feat[0, :N_NODES]
    aR = feat[1, :N_NODES]
    return _tc_call(x, aF, deg[0], aR, deg[1],
                    W_l1.T, W_r1.T, W_l2.T, W_r2.T,
                    b_l1.reshape(1, D), b_l2.reshape(1, D))
